# baseline (speedup 1.0000x reference)
import functools

import jax
import jax.numpy as jnp
from jax import lax
from jax.experimental import pallas as pl
from jax.experimental.pallas import tpu as pltpu
from jax.experimental.pallas import tpu_sc as plsc

NC = 2
NS = 16
NW = NC * NS
CH = 128
D = 128


def _sc_mesh():
    return plsc.VectorSubcoreMesh(
        core_axis_name="c", subcore_axis_name="s", num_cores=NC, num_subcores=NS
    )


def _sc_degree(dst_p, zeros, ones128, n_acc, cpt):
    rpt = n_acc // NS

    @functools.partial(
        pl.kernel,
        out_type=jax.ShapeDtypeStruct((NC * n_acc, D), jnp.float32),
        mesh=_sc_mesh(),
        scratch_types=[
            pltpu.VMEM((CH,), jnp.int32),
            pltpu.VMEM((CH, D), jnp.float32),
            pltpu.VMEM_SHARED((n_acc, D), jnp.float32),
        ],
    )
    def deg_kernel(dst_hbm, z_hbm, ones_hbm, out_hbm, idx_v, ones_v, acc_sh):
        cid = lax.axis_index("c")
        sid = lax.axis_index("s")
        wid = cid * NS + sid
        r0 = sid * rpt

        pltpu.sync_copy(ones_hbm, ones_v)
        pltpu.sync_copy(z_hbm.at[pl.ds(r0, rpt)], acc_sh.at[pl.ds(r0, rpt)])
        plsc.subcore_barrier()

        @pl.loop(0, cpt)
        def _(k):
            base = (wid * cpt + k) * CH
            pltpu.sync_copy(dst_hbm.at[pl.ds(base, CH)], idx_v)
            pltpu.sync_copy(ones_v, acc_sh.at[idx_v], add=True)

        plsc.subcore_barrier()
        pltpu.sync_copy(
            acc_sh.at[pl.ds(r0, rpt)], out_hbm.at[pl.ds(cid * n_acc + r0, rpt)]
        )

    return deg_kernel(dst_p, zeros, ones128)


def _sc_aggregate(hs, src_p, dst2d, zeros, n_acc, cpt):
    rpt = n_acc // NS
    hc = cpt // 2

    @functools.partial(
        pl.kernel,
        out_type=jax.ShapeDtypeStruct((NC * n_acc, D), jnp.float32),
        mesh=_sc_mesh(),
        scratch_types=[
            pltpu.VMEM((hc * CH,), jnp.int32),
            pltpu.VMEM((hc, CH), jnp.int32),
            pltpu.VMEM((CH, D), jnp.float32),
            pltpu.VMEM((CH, D), jnp.float32),
            pltpu.SemaphoreType.DMA,
            pltpu.SemaphoreType.DMA,
            pltpu.VMEM_SHARED((n_acc, D), jnp.float32),
        ],
    )
    def agg_kernel(hs_hbm, src_hbm, dst_hbm, z_hbm, out_hbm,
                   srcv, dstv, rows0, rows1, sem0, sem1, acc_sh):
        cid = lax.axis_index("c")
        sid = lax.axis_index("s")
        wid = cid * NS + sid
        r0 = sid * rpt

        pltpu.sync_copy(z_hbm.at[pl.ds(r0, rpt)], acc_sh.at[pl.ds(r0, rpt)])
        plsc.subcore_barrier()

        def gather(k, buf, sem):
            return pltpu.async_copy(hs_hbm.at[srcv.at[pl.ds(k * CH, CH)]], buf, sem)

        def step(k, buf, sem):
            pltpu.make_async_copy(
                hs_hbm.at[srcv.at[pl.ds(k * CH, CH)]], buf, sem
            ).wait()
            pltpu.sync_copy(buf, acc_sh.at[dstv.at[k]], add=True)

            @pl.when(k + 2 < hc)
            def _():
                gather(k + 2, buf, sem)

        for p in range(2):
            pltpu.sync_copy(
                src_hbm.at[pl.ds((wid * cpt + p * hc) * CH, hc * CH)], srcv
            )
            pltpu.sync_copy(dst_hbm.at[pl.ds(wid * cpt + p * hc, hc)], dstv)
            gather(0, rows0, sem0)
            gather(1, rows1, sem1)

            @pl.loop(0, hc // 2)
            def _(kk):
                step(2 * kk, rows0, sem0)
                step(2 * kk + 1, rows1, sem1)

        plsc.subcore_barrier()
        pltpu.sync_copy(
            acc_sh.at[pl.ds(r0, rpt)], out_hbm.at[pl.ds(cid * n_acc + r0, rpt)]
        )

    return agg_kernel(hs, src_p, dst2d, zeros)


_BLK = 2000


def _tc_first(x, W, p0, p1):
    n = x.shape[0]

    def body(x_ref, w_ref, p0_ref, p1_ref, hs_ref, dinv_ref):
        deg = p0_ref[:, 0:1] + p1_ref[:, 0:1] + 1.0
        dinv = lax.rsqrt(deg)
        h = jnp.dot(
            x_ref[...], w_ref[...],
            preferred_element_type=jnp.float32, precision=lax.Precision.HIGHEST,
        )
        hs_ref[...] = h * dinv
        dinv_ref[...] = dinv

    return pl.pallas_call(
        body,
        grid=(n // _BLK,),
        in_specs=[
            pl.BlockSpec((_BLK, D), lambda i: (i, 0)),
            pl.BlockSpec((D, D), lambda i: (0, 0)),
            pl.BlockSpec((_BLK, D), lambda i: (i, 0)),
            pl.BlockSpec((_BLK, D), lambda i: (i, 0)),
        ],
        out_specs=[
            pl.BlockSpec((_BLK, D), lambda i: (i, 0)),
            pl.BlockSpec((_BLK, 1), lambda i: (i, 0)),
        ],
        out_shape=[
            jax.ShapeDtypeStruct((n, D), jnp.float32),
            jax.ShapeDtypeStruct((n, 1), jnp.float32),
        ],
    )(x, W, p0, p1)


def _tc_mid(a0, a1, hs, dinv, b, W):
    n = hs.shape[0]

    def body(a0_ref, a1_ref, hs_ref, dinv_ref, b_ref, w_ref, o_ref):
        dinv = dinv_ref[...]
        z = jnp.maximum(
            dinv * (a0_ref[...] + a1_ref[...] + hs_ref[...]) + b_ref[...], 0.0
        )
        o_ref[...] = (
            jnp.dot(z, w_ref[...], preferred_element_type=jnp.float32,
                    precision=lax.Precision.HIGHEST)
            * dinv
        )

    return pl.pallas_call(
        body,
        grid=(n // _BLK,),
        in_specs=[
            pl.BlockSpec((_BLK, D), lambda i: (i, 0)),
            pl.BlockSpec((_BLK, D), lambda i: (i, 0)),
            pl.BlockSpec((_BLK, D), lambda i: (i, 0)),
            pl.BlockSpec((_BLK, 1), lambda i: (i, 0)),
            pl.BlockSpec((1, D), lambda i: (0, 0)),
            pl.BlockSpec((D, D), lambda i: (0, 0)),
        ],
        out_specs=pl.BlockSpec((_BLK, D), lambda i: (i, 0)),
        out_shape=jax.ShapeDtypeStruct((n, D), jnp.float32),
    )(a0, a1, hs, dinv, b, W)


def _tc_final(a0, a1, hs, dinv, b):
    n = hs.shape[0]

    def body(a0_ref, a1_ref, hs_ref, dinv_ref, b_ref, o_ref):
        o_ref[...] = (
            dinv_ref[...] * (a0_ref[...] + a1_ref[...] + hs_ref[...]) + b_ref[...]
        )

    return pl.pallas_call(
        body,
        grid=(n // _BLK,),
        in_specs=[
            pl.BlockSpec((_BLK, D), lambda i: (i, 0)),
            pl.BlockSpec((_BLK, D), lambda i: (i, 0)),
            pl.BlockSpec((_BLK, D), lambda i: (i, 0)),
            pl.BlockSpec((_BLK, 1), lambda i: (i, 0)),
            pl.BlockSpec((1, D), lambda i: (0, 0)),
        ],
        out_specs=pl.BlockSpec((_BLK, D), lambda i: (i, 0)),
        out_shape=jax.ShapeDtypeStruct((n, D), jnp.float32),
    )(a0, a1, hs, dinv, b)


def kernel(x, edge_index, W1, b1, W2, b2, W3, b3):
    n = x.shape[0]
    e = edge_index.shape[1]

    e_pad = -(-e // (NW * CH * 4)) * (NW * CH * 4)
    cpt = e_pad // (NW * CH)
    n_acc = -(-(n + 1) // (NS * 8)) * (NS * 8)
    src_p = jnp.concatenate(
        [edge_index[0].astype(jnp.int32), jnp.zeros((e_pad - e,), jnp.int32)]
    )
    dst_p = jnp.concatenate(
        [edge_index[1].astype(jnp.int32), jnp.full((e_pad - e,), n, jnp.int32)]
    )
    dst2d = dst_p.reshape(NW * cpt, CH)

    zeros = jnp.zeros((n_acc, D), jnp.float32)
    ones128 = jnp.ones((CH, D), jnp.float32)

    deg_parts = _sc_degree(dst_p, zeros, ones128, n_acc, cpt)
    p0 = deg_parts[:n]
    p1 = deg_parts[n_acc : n_acc + n]

    h1s, dinv = _tc_first(x, W1, p0, p1)

    b1r = b1.reshape(1, D)
    b2r = b2.reshape(1, D)
    b3r = b3.reshape(1, D)

    agg = _sc_aggregate(h1s, src_p, dst2d, zeros, n_acc, cpt)
    h2s = _tc_mid(agg[:n], agg[n_acc : n_acc + n], h1s, dinv, b1r, W2)

    agg = _sc_aggregate(h2s, src_p, dst2d, zeros, n_acc, cpt)
    h3s = _tc_mid(agg[:n], agg[n_acc : n_acc + n], h2s, dinv, b2r, W3)

    agg = _sc_aggregate(h3s, src_p, dst2d, zeros, n_acc, cpt)
    return _tc_final(agg[:n], agg[n_acc : n_acc + n], h3s, dinv, b3r)


# speedup vs baseline: 1.2271x; 1.2271x over previous
import functools

import jax
import jax.numpy as jnp
from jax import lax
from jax.experimental import pallas as pl
from jax.experimental.pallas import tpu as pltpu
from jax.experimental.pallas import tpu_sc as plsc

NC = 2
NS = 16
NW = NC * NS
CH = 128
D = 128


def _sc_mesh():
    return plsc.VectorSubcoreMesh(
        core_axis_name="c", subcore_axis_name="s", num_cores=NC, num_subcores=NS
    )


def _sc_degree(dst_p, zeros, ones128, n_acc, cpt):
    rpt = n_acc // NS

    @functools.partial(
        pl.kernel,
        out_type=jax.ShapeDtypeStruct((NC * n_acc, D), jnp.float32),
        mesh=_sc_mesh(),
        scratch_types=[
            pltpu.VMEM((CH,), jnp.int32),
            pltpu.VMEM((CH, D), jnp.float32),
            pltpu.VMEM_SHARED((n_acc, D), jnp.float32),
        ],
    )
    def deg_kernel(dst_hbm, z_hbm, ones_hbm, out_hbm, idx_v, ones_v, acc_sh):
        cid = lax.axis_index("c")
        sid = lax.axis_index("s")
        wid = cid * NS + sid
        r0 = sid * rpt

        pltpu.sync_copy(ones_hbm, ones_v)
        pltpu.sync_copy(z_hbm.at[pl.ds(r0, rpt)], acc_sh.at[pl.ds(r0, rpt)])
        plsc.subcore_barrier()

        @pl.loop(0, cpt)
        def _(k):
            base = (wid * cpt + k) * CH
            pltpu.sync_copy(dst_hbm.at[pl.ds(base, CH)], idx_v)
            pltpu.sync_copy(ones_v, acc_sh.at[idx_v], add=True)

        plsc.subcore_barrier()
        pltpu.sync_copy(
            acc_sh.at[pl.ds(r0, rpt)], out_hbm.at[pl.ds(cid * n_acc + r0, rpt)]
        )

    return deg_kernel(dst_p, zeros, ones128)


def _sc_aggregate(hs, src_p, dst_p, zeros, n_acc, cpt0, cpt1):
    rpt = n_acc // NS

    @functools.partial(
        pl.kernel,
        out_type=jax.ShapeDtypeStruct((NC * n_acc, D), jnp.float32),
        mesh=_sc_mesh(),
        scratch_types=[
            pltpu.VMEM((CH,), jnp.int32),
            pltpu.VMEM((CH,), jnp.int32),
            pltpu.VMEM((CH, D), jnp.float32),
            pltpu.VMEM_SHARED((n_acc, D), jnp.float32),
            pltpu.SemaphoreType.DMA,
        ],
    )
    def agg_kernel(hs_hbm, src_hbm, dst_hbm, z_hbm, out_hbm, srcv, dstv, rows_v, acc_sh, sem):
        cid = lax.axis_index("c")
        sid = lax.axis_index("s")
        r0 = sid * rpt

        pltpu.sync_copy(z_hbm.at[pl.ds(r0, rpt)], acc_sh.at[pl.ds(r0, rpt)])
        plsc.subcore_barrier()

        my_cpt = jnp.where(cid == 0, cpt0, cpt1)
        tile_base = jnp.where(cid == 0, sid * cpt0, NS * cpt0 + sid * cpt1)

        @pl.loop(0, max(cpt0, cpt1))
        def _(k):
            @pl.when(k < my_cpt)
            def _():
                base = (tile_base + k) * CH
                pltpu.sync_copy(src_hbm.at[pl.ds(base, CH)], srcv)
                pltpu.sync_copy(dst_hbm.at[pl.ds(base, CH)], dstv)
                pltpu.async_copy(hs_hbm.at[srcv], rows_v, sem).wait()
                pltpu.sync_copy(rows_v, acc_sh.at[dstv], add=True)

        plsc.subcore_barrier()
        pltpu.sync_copy(
            acc_sh.at[pl.ds(r0, rpt)], out_hbm.at[pl.ds(cid * n_acc + r0, rpt)]
        )

    return agg_kernel(hs, src_p, dst_p, zeros)


_BLK = 2000


def _tc_first(x, W, p0, p1):
    n = x.shape[0]

    def body(x_ref, w_ref, p0_ref, p1_ref, hs_ref, dinv_ref):
        deg = p0_ref[:, 0:1] + p1_ref[:, 0:1] + 1.0
        dinv = lax.rsqrt(deg)
        h = jnp.dot(
            x_ref[...], w_ref[...],
            preferred_element_type=jnp.float32, precision=lax.Precision.HIGHEST,
        )
        hs_ref[...] = h * dinv
        dinv_ref[...] = dinv

    return pl.pallas_call(
        body,
        grid=(n // _BLK,),
        in_specs=[
            pl.BlockSpec((_BLK, D), lambda i: (i, 0)),
            pl.BlockSpec((D, D), lambda i: (0, 0)),
            pl.BlockSpec((_BLK, D), lambda i: (i, 0)),
            pl.BlockSpec((_BLK, D), lambda i: (i, 0)),
        ],
        out_specs=[
            pl.BlockSpec((_BLK, D), lambda i: (i, 0)),
            pl.BlockSpec((_BLK, 1), lambda i: (i, 0)),
        ],
        out_shape=[
            jax.ShapeDtypeStruct((n, D), jnp.float32),
            jax.ShapeDtypeStruct((n, 1), jnp.float32),
        ],
    )(x, W, p0, p1)


def _tc_mid(a0, a1, hs, dinv, b, W):
    n = hs.shape[0]

    def body(a0_ref, a1_ref, hs_ref, dinv_ref, b_ref, w_ref, o_ref):
        dinv = dinv_ref[...]
        z = jnp.maximum(
            dinv * (a0_ref[...] + a1_ref[...] + hs_ref[...]) + b_ref[...], 0.0
        )
        o_ref[...] = (
            jnp.dot(z, w_ref[...], preferred_element_type=jnp.float32,
                    precision=lax.Precision.HIGHEST)
            * dinv
        )

    return pl.pallas_call(
        body,
        grid=(n // _BLK,),
        in_specs=[
            pl.BlockSpec((_BLK, D), lambda i: (i, 0)),
            pl.BlockSpec((_BLK, D), lambda i: (i, 0)),
            pl.BlockSpec((_BLK, D), lambda i: (i, 0)),
            pl.BlockSpec((_BLK, 1), lambda i: (i, 0)),
            pl.BlockSpec((1, D), lambda i: (0, 0)),
            pl.BlockSpec((D, D), lambda i: (0, 0)),
        ],
        out_specs=pl.BlockSpec((_BLK, D), lambda i: (i, 0)),
        out_shape=jax.ShapeDtypeStruct((n, D), jnp.float32),
    )(a0, a1, hs, dinv, b, W)


def _tc_final(a0, a1, hs, dinv, b):
    n = hs.shape[0]

    def body(a0_ref, a1_ref, hs_ref, dinv_ref, b_ref, o_ref):
        o_ref[...] = (
            dinv_ref[...] * (a0_ref[...] + a1_ref[...] + hs_ref[...]) + b_ref[...]
        )

    return pl.pallas_call(
        body,
        grid=(n // _BLK,),
        in_specs=[
            pl.BlockSpec((_BLK, D), lambda i: (i, 0)),
            pl.BlockSpec((_BLK, D), lambda i: (i, 0)),
            pl.BlockSpec((_BLK, D), lambda i: (i, 0)),
            pl.BlockSpec((_BLK, 1), lambda i: (i, 0)),
            pl.BlockSpec((1, D), lambda i: (0, 0)),
        ],
        out_specs=pl.BlockSpec((_BLK, D), lambda i: (i, 0)),
        out_shape=jax.ShapeDtypeStruct((n, D), jnp.float32),
    )(a0, a1, hs, dinv, b)


def kernel(x, edge_index, W1, b1, W2, b2, W3, b3):
    n = x.shape[0]
    e = edge_index.shape[1]

    e_pad = -(-e // (NW * CH)) * (NW * CH)
    cpt = e_pad // (NW * CH)
    cpt0 = (2 * cpt * 49 + 40) // 80
    cpt1 = 2 * cpt - cpt0
    n_acc = -(-(n + 1) // (NS * 8)) * (NS * 8)
    src_p = jnp.concatenate(
        [edge_index[0].astype(jnp.int32), jnp.zeros((e_pad - e,), jnp.int32)]
    )
    dst_p = jnp.concatenate(
        [edge_index[1].astype(jnp.int32), jnp.full((e_pad - e,), n, jnp.int32)]
    )


    zeros = jnp.zeros((n_acc, D), jnp.float32)
    ones128 = jnp.ones((CH, D), jnp.float32)

    deg_parts = _sc_degree(dst_p, zeros, ones128, n_acc, cpt)
    p0 = deg_parts[:n]
    p1 = deg_parts[n_acc : n_acc + n]

    h1s, dinv = _tc_first(x, W1, p0, p1)

    b1r = b1.reshape(1, D)
    b2r = b2.reshape(1, D)
    b3r = b3.reshape(1, D)

    agg = _sc_aggregate(h1s, src_p, dst_p, zeros, n_acc, cpt0, cpt1)
    h2s = _tc_mid(agg[:n], agg[n_acc : n_acc + n], h1s, dinv, b1r, W2)

    agg = _sc_aggregate(h2s, src_p, dst_p, zeros, n_acc, cpt0, cpt1)
    h3s = _tc_mid(agg[:n], agg[n_acc : n_acc + n], h2s, dinv, b2r, W3)

    agg = _sc_aggregate(h3s, src_p, dst_p, zeros, n_acc, cpt0, cpt1)
    return _tc_final(agg[:n], agg[n_acc : n_acc + n], h3s, dinv, b3r)
